# baseline (speedup 1.0000x reference)
import jax
import jax.numpy as jnp
from jax import lax
from jax.experimental import pallas as pl
from jax.experimental.pallas import tpu as pltpu
from jax.experimental.pallas import tpu_sc as plsc

HOTN = 32768
HASH_SIZE = 500000
EMB_DIM = 32
BATCH = 16384

_NC = 2
_NS = 16
_NW = _NC * _NS
_BPW = BATCH // _NW
_NVEC = _BPW // 16
_CH = 32
_NCH = _BPW // _CH


def _sc_body(inp_hbm, wh_hbm, whash_hbm, out_hbm,
             raw_v, sg_v, sr_v, m_v, grp_b, out_b, sem):
    wid = lax.axis_index("s") * _NC + lax.axis_index("c")
    base = wid * _BPW

    pltpu.sync_copy(inp_hbm.at[pl.ds(base, _BPW)], raw_v)

    for i in range(_NVEC):
        v = raw_v[pl.ds(i * 16, 16)]
        s = (v & 31) + ((v >> 5) & 31) + ((v >> 10) & 31) + ((v >> 15) & 31)
        s = (s & 31) + (s >> 5)
        hot = jnp.logical_or(s == 0, s == 31)
        rh = v & (HOTN - 1)
        rc = jnp.where(v >= HASH_SIZE, v - HASH_SIZE, v)
        r = jnp.where(hot, rh, rc)
        m_v[pl.ds(i * 16, 16)] = jnp.where(hot, 1, 0).astype(jnp.int32)
        sg_v[pl.ds(i * 16, 16)] = r >> 3
        sr_v[pl.ds(i * 16, 16)] = r & 7

    def fire(ch1, p):
        for h in range(_CH // 16):
            sg16 = sg_v[pl.ds(ch1 * _CH + h * 16, 16)]
            m16 = m_v[pl.ds(ch1 * _CH + h * 16, 16)]
            for j in range(16):
                i = h * 16 + j
                g = sg16[j]
                hotf = m16[j]

                @pl.when(hotf == 1)
                def _():
                    pltpu.async_copy(wh_hbm.at[g], grp_b.at[p, i], sem.at[p])

                @pl.when(hotf == 0)
                def _():
                    pltpu.async_copy(whash_hbm.at[g], grp_b.at[p, i],
                                     sem.at[p])

    def drain(p):
        pltpu.make_async_copy(
            whash_hbm.at[pl.ds(0, _CH)], grp_b.at[p], sem.at[p]).wait()

    def extract(ch, p):
        for h in range(_CH // 16):
            sr16 = sr_v[pl.ds(ch * _CH + h * 16, 16)]
            for j in range(16):
                i = h * 16 + j
                r = sr16[j]
                orow = ch * 8 + (i >> 2)
                ocol = (i & 3) * 32
                for c0 in (0, 16):
                    out_b[orow, pl.ds(ocol + c0, 16)] = \
                        grp_b[p, i, r, pl.ds(c0, 16)]

    fire(0, 0)
    fire(1, 1)

    def chunk(ch, _):
        for p in (0, 1, 2):
            @pl.when(lax.rem(ch, 3) == p)
            def _():
                @pl.when(ch + 2 < _NCH)
                def _():
                    fire(ch + 2, (p + 2) % 3)
                drain(p)
                extract(ch, p)
        return 0

    lax.fori_loop(0, _NCH, chunk, 0)
    pltpu.sync_copy(out_b, out_hbm.at[pl.ds(wid * 128, 128)])


@jax.jit
def _run(inp, wh, whash):
    mesh = plsc.VectorSubcoreMesh(core_axis_name="c", subcore_axis_name="s")
    f = pl.kernel(
        _sc_body,
        out_type=jax.ShapeDtypeStruct((BATCH // 4, 128), jnp.float32),
        mesh=mesh,
        compiler_params=pltpu.CompilerParams(use_tc_tiling_on_sc=True),
        scratch_types=[
            pltpu.VMEM((_BPW,), jnp.int32),
            pltpu.VMEM((_BPW,), jnp.int32),
            pltpu.VMEM((_BPW,), jnp.int32),
            pltpu.VMEM((_BPW,), jnp.int32),
            pltpu.VMEM((3, _CH, 8, 32), jnp.float32),
            pltpu.VMEM((128, 128), jnp.float32),
            pltpu.SemaphoreType.DMA((3,)),
        ],
    )
    return f(inp, wh, whash)


def kernel(input, offsets, weight_h, weight_hash):
    del offsets
    wh = weight_h.reshape(HOTN // 8, 8, EMB_DIM)
    whash = weight_hash.reshape(HASH_SIZE // 8, 8, EMB_DIM)
    out = _run(input.astype(jnp.int32), wh, whash)
    return out.reshape(BATCH, EMB_DIM)
